# baseline (speedup 1.0000x reference)
import functools

import jax
import jax.numpy as jnp
from jax import lax
from jax.experimental import pallas as pl
from jax.experimental.pallas import tpu as pltpu
from jax.experimental.pallas import tpu_sc as plsc

SEQ = 50
BATCH = 1024
EMBED = 300
HIDDEN = 100
GP = 128
G4 = 4 * GP
NUM_CLASS = 10
TOTAL = SEQ * BATCH

_NC = 2
_NS = 16
_NW = _NC * _NS
_PER_W = TOTAL // _NW
_CH = 80
_NCH = _PER_W // _CH
EPAD = 384


def _sc_gather_body(table_hbm, idx_hbm, out_hbm, idx_v, rows0, rows1, sem0,
                    sem1):
    wid = lax.axis_index("s") * _NC + lax.axis_index("c")
    base = wid * _PER_W
    pltpu.sync_copy(idx_hbm.at[pl.ds(base, _PER_W)], idx_v)
    bufs = (rows0, rows1)
    sems = (sem0, sem1)

    def _gather(j, buf, sem):
        return pltpu.make_async_copy(
            table_hbm.at[idx_v.at[pl.ds(j * _CH, _CH)]], buf, sem)

    _gather(0, bufs[0], sems[0]).start()
    for j in range(_NCH):
        cur, csem = bufs[j % 2], sems[j % 2]
        if j + 1 < _NCH:
            _gather(j + 1, bufs[(j + 1) % 2], sems[(j + 1) % 2]).start()
        _gather(j, cur, csem).wait()
        pltpu.sync_copy(cur, out_hbm.at[pl.ds(base + j * _CH, _CH)])


def _sc_gather(emb384, idx):
    mesh = plsc.VectorSubcoreMesh(core_axis_name="c", subcore_axis_name="s")
    call = pl.kernel(
        _sc_gather_body,
        out_type=jax.ShapeDtypeStruct((TOTAL, EPAD), jnp.float32),
        mesh=mesh,
        scratch_types=[
            pltpu.VMEM((_PER_W,), jnp.int32),
            pltpu.VMEM((_CH, EPAD), jnp.float32),
            pltpu.VMEM((_CH, EPAD), jnp.float32),
            pltpu.SemaphoreType.DMA,
            pltpu.SemaphoreType.DMA,
        ],
    )
    return call(emb384, idx)


_TB = 4096
VOCAB_N = 100000


def _transpose_body(xt_ref, o_ref):
    xp = jnp.concatenate(
        [xt_ref[...], jnp.zeros((EPAD - EMBED, _TB), jnp.float32)], axis=0)
    o_ref[...] = xp.T


def _transpose_table(embT):
    grid = (VOCAB_N + _TB - 1) // _TB
    return pl.pallas_call(
        _transpose_body,
        grid=(grid,),
        in_specs=[pl.BlockSpec((EMBED, _TB), lambda i: (0, i))],
        out_specs=pl.BlockSpec((_TB, EPAD), lambda i: (i, 0)),
        out_shape=jax.ShapeDtypeStruct((VOCAB_N, EPAD), jnp.float32),
        compiler_params=pltpu.CompilerParams(
            dimension_semantics=("arbitrary",)),
    )(embT)


def _layer_body(nparts, last, *refs):
    i = 0
    xf = refs[i:i + nparts]; i += nparts
    xb = refs[i:i + nparts]; i += nparts
    wf = refs[i:i + nparts]; i += nparts
    wb = refs[i:i + nparts]; i += nparts
    whhf, whhb, bf, bb = refs[i:i + 4]; i += 4
    if last:
        h00, h01, h10, h11, fcw, fcb = refs[i:i + 6]; i += 6
        ysf, ysb, hfo, hbo, fco = refs[i:i + 5]; i += 5
    else:
        ysf, ysb, hfo, hbo = refs[i:i + 4]; i += 4
    hf, cf, hb, cb = refs[i:i + 4]

    t = pl.program_id(0)

    @pl.when(t == 0)
    def _():
        hf[...] = jnp.zeros_like(hf)
        cf[...] = jnp.zeros_like(cf)
        hb[...] = jnp.zeros_like(hb)
        cb[...] = jnp.zeros_like(cb)

    def _dot(a, b):
        return jnp.dot(a, b, preferred_element_type=jnp.float32)

    def _step(xparts, wparts, whh, b, h_s, c_s):
        g = _dot(h_s[...], whh[...])
        for xp, wp in zip(xparts, wparts):
            g = g + _dot(xp[0].astype(jnp.bfloat16), wp[...])
        g = g + b[...]
        ii = jax.nn.sigmoid(g[:, 0:GP])
        ff = jax.nn.sigmoid(g[:, GP:2 * GP])
        gg = jnp.tanh(g[:, 2 * GP:3 * GP])
        oo = jax.nn.sigmoid(g[:, 3 * GP:4 * GP])
        c2 = ff * c_s[...] + ii * gg
        h2 = oo * jnp.tanh(c2)
        h_s[...] = h2.astype(jnp.bfloat16)
        c_s[...] = c2
        return h2

    h2f = _step(xf, wf, whhf, bf, hf, cf)
    h2b = _step(xb, wb, whhb, bb, hb, cb)
    ysf[0] = h2f[:, :HIDDEN]
    ysb[0] = h2b[:, :HIDDEN]

    @pl.when(t == SEQ - 1)
    def _():
        hfo[...] = h2f[:, :HIDDEN]
        hbo[...] = h2b[:, :HIDDEN]
        if last:
            feats = (h00[...], h01[...], h10[...], h11[...],
                     h2f[:, :HIDDEN], h2b[:, :HIDDEN])
            o = fcb[...]
            for k6 in range(6):
                o = o + _dot(feats[k6], fcw[k6])
            fco[...] = o


def _pad_gates(m):
    k = m.shape[0]
    m = m.reshape(k, 4, HIDDEN)
    m = jnp.pad(m, ((0, 0), (0, 0), (0, GP - HIDDEN)))
    return m.reshape(k, G4)


def _prep_dir(p, in_splits):
    W_ih, W_hh, b_ih, b_hh = p
    wt = _pad_gates(W_ih.T)
    need = sum(in_splits)
    if need > wt.shape[0]:
        wt = jnp.pad(wt, ((0, need - wt.shape[0]), (0, 0)))
    parts = []
    off = 0
    for s in in_splits:
        parts.append(wt[off:off + s].astype(jnp.bfloat16))
        off += s
    whh = _pad_gates(W_hh.T)
    whh = jnp.pad(whh, ((0, GP - HIDDEN), (0, 0)))
    whh = whh.astype(jnp.bfloat16)
    bias = _pad_gates((b_ih + b_hh)[None, :])
    return parts, whh, bias


def _full_spec(a):
    nd = a.ndim
    return pl.BlockSpec(a.shape, lambda t, _n=nd: (0,) * _n)


def _run_layer(xs_f, xs_b, layer, in_splits, fc_extra=None):
    nparts = len(xs_f)
    last = fc_extra is not None
    wfp, whhf, bf = _prep_dir(layer[0], in_splits)
    wbp, whhb, bb = _prep_dir(layer[1], in_splits)

    ins = []
    in_specs = []
    for a in xs_f:
        ins.append(a)
        in_specs.append(pl.BlockSpec((1, BATCH, a.shape[2]),
                                     lambda t: (t, 0, 0)))
    for a in xs_b:
        ins.append(a)
        in_specs.append(pl.BlockSpec((1, BATCH, a.shape[2]),
                                     lambda t: (SEQ - 1 - t, 0, 0)))
    for w in wfp + wbp + [whhf, whhb, bf, bb]:
        ins.append(w)
        in_specs.append(_full_spec(w))
    if last:
        for a in fc_extra:
            ins.append(a)
            in_specs.append(_full_spec(a))

    out_shapes = [
        jax.ShapeDtypeStruct((SEQ, BATCH, HIDDEN), jnp.float32),
        jax.ShapeDtypeStruct((SEQ, BATCH, HIDDEN), jnp.float32),
        jax.ShapeDtypeStruct((BATCH, HIDDEN), jnp.float32),
        jax.ShapeDtypeStruct((BATCH, HIDDEN), jnp.float32),
    ]
    out_specs = [
        pl.BlockSpec((1, BATCH, HIDDEN), lambda t: (t, 0, 0)),
        pl.BlockSpec((1, BATCH, HIDDEN), lambda t: (SEQ - 1 - t, 0, 0)),
        pl.BlockSpec((BATCH, HIDDEN), lambda t: (0, 0)),
        pl.BlockSpec((BATCH, HIDDEN), lambda t: (0, 0)),
    ]
    if last:
        out_shapes.append(jax.ShapeDtypeStruct((BATCH, NUM_CLASS),
                                               jnp.float32))
        out_specs.append(pl.BlockSpec((BATCH, NUM_CLASS), lambda t: (0, 0)))

    call = pl.pallas_call(
        functools.partial(_layer_body, nparts, last),
        grid=(SEQ,),
        in_specs=in_specs,
        out_specs=out_specs,
        out_shape=out_shapes,
        scratch_shapes=[
            pltpu.VMEM((BATCH, GP), jnp.bfloat16),
            pltpu.VMEM((BATCH, GP), jnp.float32),
            pltpu.VMEM((BATCH, GP), jnp.bfloat16),
            pltpu.VMEM((BATCH, GP), jnp.float32),
        ],
        compiler_params=pltpu.CompilerParams(
            dimension_semantics=("arbitrary",)),
    )
    return call(*ins)


def _lstm_tail(E, lstm_params, fc_w, fc_b):
    fcw = fc_w.T.reshape(6, HIDDEN, NUM_CLASS)
    fcb = fc_b.reshape(1, NUM_CLASS)

    ysf, ysb, h00, h01 = _run_layer([E], [E], lstm_params[0], [EPAD])
    ysf, ysb, h10, h11 = _run_layer([ysf, ysb], [ysf, ysb],
                                    lstm_params[1], [HIDDEN, HIDDEN])
    outs = _run_layer([ysf, ysb], [ysf, ysb], lstm_params[2],
                      [HIDDEN, HIDDEN],
                      fc_extra=[h00, h01, h10, h11, fcw, fcb])
    return outs[4]


def kernel(x, emb, lstm_params, fc_w, fc_b):
    idx = x.astype(jnp.int32).reshape(TOTAL)
    emb384 = _transpose_table(emb.T)
    E = _sc_gather(emb384, idx)
    return _lstm_tail(E.reshape(SEQ, BATCH, EPAD), lstm_params, fc_w, fc_b)


# speedup vs baseline: 1.0544x; 1.0544x over previous
import functools

import jax
import jax.numpy as jnp
from jax import lax
from jax.experimental import pallas as pl
from jax.experimental.pallas import tpu as pltpu
from jax.experimental.pallas import tpu_sc as plsc

SEQ = 50
BATCH = 1024
EMBED = 300
HIDDEN = 100
GP = 128
G4 = 4 * GP
NUM_CLASS = 10
TOTAL = SEQ * BATCH

_NC = 2
_NS = 16
_NW = _NC * _NS
_PER_W = TOTAL // _NW
_CH = 80
_NCH = _PER_W // _CH
EPAD = 384


def _sc_gather_body(table_hbm, idx_hbm, out_hbm, idx_v, rows0, rows1, sem0,
                    sem1):
    wid = lax.axis_index("s") * _NC + lax.axis_index("c")
    base = wid * _PER_W
    pltpu.sync_copy(idx_hbm.at[pl.ds(base, _PER_W)], idx_v)
    bufs = (rows0, rows1)
    sems = (sem0, sem1)

    def _gather(j, buf, sem):
        return pltpu.make_async_copy(
            table_hbm.at[idx_v.at[pl.ds(j * _CH, _CH)]], buf, sem)

    _gather(0, bufs[0], sems[0]).start()
    for j in range(_NCH):
        cur, csem = bufs[j % 2], sems[j % 2]
        if j + 1 < _NCH:
            _gather(j + 1, bufs[(j + 1) % 2], sems[(j + 1) % 2]).start()
        _gather(j, cur, csem).wait()
        pltpu.sync_copy(cur, out_hbm.at[pl.ds(base + j * _CH, _CH)])


def _sc_gather(emb384, idx):
    mesh = plsc.VectorSubcoreMesh(core_axis_name="c", subcore_axis_name="s")
    call = pl.kernel(
        _sc_gather_body,
        out_type=jax.ShapeDtypeStruct((TOTAL, EPAD), jnp.float32),
        mesh=mesh,
        scratch_types=[
            pltpu.VMEM((_PER_W,), jnp.int32),
            pltpu.VMEM((_CH, EPAD), jnp.float32),
            pltpu.VMEM((_CH, EPAD), jnp.float32),
            pltpu.SemaphoreType.DMA,
            pltpu.SemaphoreType.DMA,
        ],
    )
    return call(emb384, idx)


_TB = 4096
VOCAB_N = 100000


def _transpose_body(xt_ref, o_ref):
    xp = jnp.concatenate(
        [xt_ref[...], jnp.zeros((EPAD - EMBED, _TB), jnp.float32)], axis=0)
    o_ref[...] = xp.T


def _transpose_table(embT):
    grid = (VOCAB_N + _TB - 1) // _TB
    return pl.pallas_call(
        _transpose_body,
        grid=(grid,),
        in_specs=[pl.BlockSpec((EMBED, _TB), lambda i: (0, i))],
        out_specs=pl.BlockSpec((_TB, EPAD), lambda i: (i, 0)),
        out_shape=jax.ShapeDtypeStruct((VOCAB_N, EPAD), jnp.float32),
        compiler_params=pltpu.CompilerParams(
            dimension_semantics=("arbitrary",)),
    )(embT)


def _layer_body(nparts, last, *refs):
    i = 0
    xf = refs[i:i + nparts]; i += nparts
    xb = refs[i:i + nparts]; i += nparts
    wf = refs[i:i + nparts]; i += nparts
    wb = refs[i:i + nparts]; i += nparts
    whhf, whhb, bf, bb = refs[i:i + 4]; i += 4
    if last:
        h00, h01, h10, h11, fcw, fcb = refs[i:i + 6]; i += 6
        ysf, ysb, hfo, hbo, fco = refs[i:i + 5]; i += 5
    else:
        ysf, ysb, hfo, hbo = refs[i:i + 4]; i += 4
    hf, cf, hb, cb = refs[i:i + 4]

    t = pl.program_id(0)

    @pl.when(t == 0)
    def _():
        hf[...] = jnp.zeros_like(hf)
        cf[...] = jnp.zeros_like(cf)
        hb[...] = jnp.zeros_like(hb)
        cb[...] = jnp.zeros_like(cb)

    def _dot(a, b):
        return jnp.dot(a, b, preferred_element_type=jnp.float32)

    def _step(xparts, wparts, whh, b, h_s, c_s):
        g = _dot(h_s[...], whh[...])
        for xp, wp in zip(xparts, wparts):
            g = g + _dot(xp[0].astype(jnp.bfloat16), wp[...])
        g = g + b[...]

        def _sig(v):
            return 0.5 * jnp.tanh(0.5 * v) + 0.5

        ii = _sig(g[:, 0:GP])
        ff = _sig(g[:, GP:2 * GP])
        gg = jnp.tanh(g[:, 2 * GP:3 * GP])
        oo = _sig(g[:, 3 * GP:4 * GP])
        c2 = ff * c_s[...] + ii * gg
        h2 = oo * jnp.tanh(c2)
        h_s[...] = h2.astype(jnp.bfloat16)
        c_s[...] = c2
        return h2

    h2f = _step(xf, wf, whhf, bf, hf, cf)
    h2b = _step(xb, wb, whhb, bb, hb, cb)
    ysf[0] = h2f[:, :HIDDEN]
    ysb[0] = h2b[:, :HIDDEN]

    @pl.when(t == SEQ - 1)
    def _():
        hfo[...] = h2f[:, :HIDDEN]
        hbo[...] = h2b[:, :HIDDEN]
        if last:
            feats = (h00[...], h01[...], h10[...], h11[...],
                     h2f[:, :HIDDEN], h2b[:, :HIDDEN])
            o = fcb[...]
            for k6 in range(6):
                o = o + _dot(feats[k6], fcw[k6])
            fco[...] = o


def _pad_gates(m):
    k = m.shape[0]
    m = m.reshape(k, 4, HIDDEN)
    m = jnp.pad(m, ((0, 0), (0, 0), (0, GP - HIDDEN)))
    return m.reshape(k, G4)


def _prep_dir(p, in_splits):
    W_ih, W_hh, b_ih, b_hh = p
    wt = _pad_gates(W_ih.T)
    need = sum(in_splits)
    if need > wt.shape[0]:
        wt = jnp.pad(wt, ((0, need - wt.shape[0]), (0, 0)))
    parts = []
    off = 0
    for s in in_splits:
        parts.append(wt[off:off + s].astype(jnp.bfloat16))
        off += s
    whh = _pad_gates(W_hh.T)
    whh = jnp.pad(whh, ((0, GP - HIDDEN), (0, 0)))
    whh = whh.astype(jnp.bfloat16)
    bias = _pad_gates((b_ih + b_hh)[None, :])
    return parts, whh, bias


def _full_spec(a):
    nd = a.ndim
    return pl.BlockSpec(a.shape, lambda t, _n=nd: (0,) * _n)


def _run_layer(xs_f, xs_b, layer, in_splits, fc_extra=None):
    nparts = len(xs_f)
    last = fc_extra is not None
    wfp, whhf, bf = _prep_dir(layer[0], in_splits)
    wbp, whhb, bb = _prep_dir(layer[1], in_splits)

    ins = []
    in_specs = []
    for a in xs_f:
        ins.append(a)
        in_specs.append(pl.BlockSpec((1, BATCH, a.shape[2]),
                                     lambda t: (t, 0, 0)))
    for a in xs_b:
        ins.append(a)
        in_specs.append(pl.BlockSpec((1, BATCH, a.shape[2]),
                                     lambda t: (SEQ - 1 - t, 0, 0)))
    for w in wfp + wbp + [whhf, whhb, bf, bb]:
        ins.append(w)
        in_specs.append(_full_spec(w))
    if last:
        for a in fc_extra:
            ins.append(a)
            in_specs.append(_full_spec(a))

    out_shapes = [
        jax.ShapeDtypeStruct((SEQ, BATCH, HIDDEN), jnp.float32),
        jax.ShapeDtypeStruct((SEQ, BATCH, HIDDEN), jnp.float32),
        jax.ShapeDtypeStruct((BATCH, HIDDEN), jnp.float32),
        jax.ShapeDtypeStruct((BATCH, HIDDEN), jnp.float32),
    ]
    out_specs = [
        pl.BlockSpec((1, BATCH, HIDDEN), lambda t: (t, 0, 0)),
        pl.BlockSpec((1, BATCH, HIDDEN), lambda t: (SEQ - 1 - t, 0, 0)),
        pl.BlockSpec((BATCH, HIDDEN), lambda t: (0, 0)),
        pl.BlockSpec((BATCH, HIDDEN), lambda t: (0, 0)),
    ]
    if last:
        out_shapes.append(jax.ShapeDtypeStruct((BATCH, NUM_CLASS),
                                               jnp.float32))
        out_specs.append(pl.BlockSpec((BATCH, NUM_CLASS), lambda t: (0, 0)))

    call = pl.pallas_call(
        functools.partial(_layer_body, nparts, last),
        grid=(SEQ,),
        in_specs=in_specs,
        out_specs=out_specs,
        out_shape=out_shapes,
        scratch_shapes=[
            pltpu.VMEM((BATCH, GP), jnp.bfloat16),
            pltpu.VMEM((BATCH, GP), jnp.float32),
            pltpu.VMEM((BATCH, GP), jnp.bfloat16),
            pltpu.VMEM((BATCH, GP), jnp.float32),
        ],
        compiler_params=pltpu.CompilerParams(
            dimension_semantics=("arbitrary",)),
    )
    return call(*ins)


def _lstm_tail(E, lstm_params, fc_w, fc_b):
    fcw = fc_w.T.reshape(6, HIDDEN, NUM_CLASS)
    fcb = fc_b.reshape(1, NUM_CLASS)

    ysf, ysb, h00, h01 = _run_layer([E], [E], lstm_params[0], [EPAD])
    ysf, ysb, h10, h11 = _run_layer([ysf, ysb], [ysf, ysb],
                                    lstm_params[1], [HIDDEN, HIDDEN])
    outs = _run_layer([ysf, ysb], [ysf, ysb], lstm_params[2],
                      [HIDDEN, HIDDEN],
                      fc_extra=[h00, h01, h10, h11, fcw, fcb])
    return outs[4]


def kernel(x, emb, lstm_params, fc_w, fc_b):
    idx = x.astype(jnp.int32).reshape(TOTAL)
    emb384 = _transpose_table(emb.T)
    E = _sc_gather(emb384, idx)
    return _lstm_tail(E.reshape(SEQ, BATCH, EPAD), lstm_params, fc_w, fc_b)
